# baseline (speedup 1.0000x reference)
import functools

import jax
import jax.numpy as jnp
from jax import lax
from jax.experimental import pallas as pl
from jax.experimental.pallas import tpu as pltpu
from jax.experimental.pallas import tpu_sc as plsc

NC = 2
NS = 16
NW = NC * NS
L = 16


def _pre_mm_kernel(v_ref, f_ref, w1_ref, w2_ref, b_ref, a_out, b_out):
    a_out[...] = (
        jnp.dot(v_ref[...], w1_ref[...], preferred_element_type=jnp.float32)
        + b_ref[...]
    )
    b_out[...] = jnp.dot(f_ref[...], w2_ref[...], preferred_element_type=jnp.float32)


def _comb_mm_kernel(v_ref, p0_ref, p1_ref, wc1_ref, wc2_ref, b_ref, o_ref):
    agg = p0_ref[...] + p1_ref[...]
    h = (
        jnp.dot(v_ref[...], wc1_ref[...], preferred_element_type=jnp.float32)
        + jnp.dot(agg, wc2_ref[...], preferred_element_type=jnp.float32)
        + b_ref[...]
    )
    o_ref[...] = v_ref[...] + jnp.maximum(h, 0.0)


def _make_sc_edge_kernel(n_pad, d, chunk, n_chunks, idx_block):
    rows_per_tile = n_pad // NS
    n_blocks = n_chunks // idx_block

    mesh = plsc.VectorSubcoreMesh(core_axis_name="c", subcore_axis_name="s")

    assert idx_block % 2 == 0 and n_blocks * idx_block == n_chunks

    @functools.partial(
        pl.kernel,
        out_type=jax.ShapeDtypeStruct((NC, n_pad, d), jnp.float32),
        mesh=mesh,
        scratch_types=[
            pltpu.VMEM((idx_block, chunk), jnp.int32),
            pltpu.VMEM((idx_block, chunk), jnp.int32),
            pltpu.VMEM((chunk, d), jnp.float32),
            pltpu.VMEM((chunk, d), jnp.float32),
            pltpu.VMEM_SHARED((n_pad, d), jnp.float32),
        ],
    )
    def sc_edge(a_hbm, b_hbm, snd_hbm, rcv_hbm, zero_hbm, out_hbm,
                snd_v, rcv_v, a_v, b_v, agg_sh):
        c = lax.axis_index("c")
        s = lax.axis_index("s")
        wid = c * NS + s
        own = pl.ds(s * rows_per_tile, rows_per_tile)

        def compute():
            def row_body(i, carry2):
                for j in range(d // L):
                    sl = pl.ds(j * L, L)
                    a_v[i, sl] = jnp.maximum(a_v[i, sl] + b_v[i, sl], 0.0)
                return carry2

            lax.fori_loop(0, chunk, row_body, 0, unroll=4)

        pltpu.sync_copy(zero_hbm.at[own], agg_sh.at[own])
        plsc.subcore_barrier()

        def block_body(g, carry0):
            pltpu.sync_copy(snd_hbm.at[wid, g], snd_v)
            pltpu.sync_copy(rcv_hbm.at[wid, g], rcv_v)

            def chunk_body(k, carry):
                pltpu.sync_copy(a_hbm.at[snd_v.at[k]], a_v)
                pltpu.sync_copy(b_hbm.at[rcv_v.at[k]], b_v)
                compute()
                pltpu.sync_copy(a_v, agg_sh.at[snd_v.at[k]], add=True)
                return carry

            lax.fori_loop(0, idx_block, chunk_body, 0, unroll=False)
            return carry0

        lax.fori_loop(0, n_blocks, block_body, 0, unroll=False)
        plsc.subcore_barrier()

        pltpu.sync_copy(agg_sh.at[own], out_hbm.at[c, own])

    return sc_edge


def kernel(variables, factors, senders, receivers, W_msg, b_msg, W_comb, b_comb):
    n_var, d = variables.shape
    e = senders.shape[0]

    w1 = W_msg[:d]
    w2 = W_msg[d:]
    wc1 = W_comb[:d]
    wc2 = W_comb[d:]
    bm = b_msg.reshape(1, d)
    bc = b_comb.reshape(1, d)

    n_pad = -(-n_var // (NS * 8)) * (NS * 8)

    var_pad = jnp.pad(variables, ((0, n_pad - n_var), (0, 0)))
    fac_pad = jnp.pad(factors, ((0, n_pad - factors.shape[0]), (0, 0)))

    blk_v = n_pad // 8
    blk_f = n_pad // 8
    grid8 = (8,)
    a_mat, b_mat = pl.pallas_call(
        _pre_mm_kernel,
        grid=grid8,
        in_specs=[
            pl.BlockSpec((blk_v, d), lambda i: (i, 0)),
            pl.BlockSpec((blk_f, d), lambda i: (i, 0)),
            pl.BlockSpec((d, d), lambda i: (0, 0)),
            pl.BlockSpec((d, d), lambda i: (0, 0)),
            pl.BlockSpec((1, d), lambda i: (0, 0)),
        ],
        out_specs=[
            pl.BlockSpec((blk_v, d), lambda i: (i, 0)),
            pl.BlockSpec((blk_f, d), lambda i: (i, 0)),
        ],
        out_shape=[
            jax.ShapeDtypeStruct((n_pad, d), jnp.float32),
            jax.ShapeDtypeStruct((n_pad, d), jnp.float32),
        ],
    )(var_pad, fac_pad, w1, w2, bm)

    chunk = 80
    per_w = e // NW
    n_chunks = per_w // chunk
    snd = senders.reshape(NW, n_chunks, chunk)
    rcv = receivers.reshape(NW, n_chunks, chunk)
    if n_chunks % 2:
        n_chunks += 1
        pad_snd = jnp.full((NW, 1, chunk), n_var, jnp.int32)
        snd = jnp.concatenate([snd, pad_snd], axis=1)
        rcv = jnp.concatenate([rcv, jnp.zeros((NW, 1, chunk), jnp.int32)], axis=1)

    idx_block = 18
    n_blocks = n_chunks // idx_block
    snd = snd.reshape(NW, n_blocks, idx_block, chunk)
    rcv = rcv.reshape(NW, n_blocks, idx_block, chunk)

    zeros = jnp.zeros((n_pad, d), jnp.float32)

    partial = _make_sc_edge_kernel(n_pad, d, chunk, n_chunks, idx_block)(
        a_mat, b_mat, snd, rcv, zeros)
    partial = partial[:, :n_var, :]

    blk = 2000
    grid = (n_var // blk,)
    row_spec = pl.BlockSpec((blk, d), lambda i: (i, 0))
    w_spec = pl.BlockSpec((d, d), lambda i: (0, 0))
    b_spec = pl.BlockSpec((1, d), lambda i: (0, 0))
    out = pl.pallas_call(
        _comb_mm_kernel,
        grid=grid,
        in_specs=[row_spec, row_spec, row_spec, w_spec, w_spec, b_spec],
        out_specs=row_spec,
        out_shape=jax.ShapeDtypeStruct((n_var, d), jnp.float32),
    )(variables, partial[0], partial[1], wc1, wc2, bc)

    return out


# speedup vs baseline: 1.4184x; 1.4184x over previous
import functools

import jax
import jax.numpy as jnp
from jax import lax
from jax.experimental import pallas as pl
from jax.experimental.pallas import tpu as pltpu
from jax.experimental.pallas import tpu_sc as plsc

NC = 2
NS = 16
NW = NC * NS
L = 16


def _pre_mm_kernel(v_ref, f_ref, w1_ref, w2_ref, b_ref, a_out, b_out):
    a_out[...] = (
        jnp.dot(v_ref[...], w1_ref[...], preferred_element_type=jnp.float32)
        + b_ref[...]
    )
    b_out[...] = jnp.dot(f_ref[...], w2_ref[...], preferred_element_type=jnp.float32)


def _comb_mm_kernel(v_ref, p0_ref, p1_ref, wc1_ref, wc2_ref, b_ref, o_ref):
    agg = p0_ref[...] + p1_ref[...]
    h = (
        jnp.dot(v_ref[...], wc1_ref[...], preferred_element_type=jnp.float32)
        + jnp.dot(agg, wc2_ref[...], preferred_element_type=jnp.float32)
        + b_ref[...]
    )
    o_ref[...] = v_ref[...] + jnp.maximum(h, 0.0)


def _make_sc_edge_kernel(n_pad, d, chunk, n_chunks, idx_block):
    rows_per_tile = n_pad // NS
    n_blocks = n_chunks // idx_block

    mesh = plsc.VectorSubcoreMesh(core_axis_name="c", subcore_axis_name="s")

    assert idx_block % 2 == 0 and n_blocks * idx_block == n_chunks

    @functools.partial(
        pl.kernel,
        out_type=jax.ShapeDtypeStruct((NC, n_pad, d), jnp.float32),
        mesh=mesh,
        scratch_types=[
            pltpu.VMEM((idx_block, chunk), jnp.int32),
            pltpu.VMEM((idx_block, chunk), jnp.int32),
            pltpu.VMEM((chunk, d), jnp.float32),
            pltpu.VMEM((chunk, d), jnp.float32),
            pltpu.VMEM((chunk, d), jnp.float32),
            pltpu.VMEM((chunk, d), jnp.float32),
            pltpu.VMEM_SHARED((n_pad, d), jnp.float32),
            pltpu.SemaphoreType.DMA,
            pltpu.SemaphoreType.DMA,
            pltpu.SemaphoreType.DMA,
        ],
    )
    def sc_edge(a_hbm, b_hbm, snd_hbm, rcv_hbm, zero_hbm, out_hbm,
                snd_v, rcv_v, a_v0, b_v0, a_v1, b_v1, agg_sh,
                sg0, sg1, ss):
        c = lax.axis_index("c")
        s = lax.axis_index("s")
        wid = c * NS + s
        own = pl.ds(s * rows_per_tile, rows_per_tile)

        def gather(k, a_v, b_v, sem):
            pltpu.async_copy(a_hbm.at[snd_v.at[k]], a_v, sem)
            pltpu.async_copy(b_hbm.at[rcv_v.at[k]], b_v, sem)

        def gather_wait(a_v, b_v, sem):
            pltpu.make_async_copy(a_hbm.at[pl.ds(0, chunk)], a_v, sem).wait()
            pltpu.make_async_copy(b_hbm.at[pl.ds(0, chunk)], b_v, sem).wait()

        def scatter_wait(a_v):
            pltpu.make_async_copy(a_hbm.at[pl.ds(0, chunk)], a_v, ss).wait()

        def compute(a_v, b_v):
            def row_body(i, carry2):
                for j in range(d // L):
                    sl = pl.ds(j * L, L)
                    a_v[i, sl] = jnp.maximum(a_v[i, sl] + b_v[i, sl], 0.0)
                return carry2

            lax.fori_loop(0, chunk, row_body, 0, unroll=4)

        pltpu.sync_copy(zero_hbm.at[own], agg_sh.at[own])
        plsc.subcore_barrier()

        def block_body(g, carry0):
            pltpu.sync_copy(snd_hbm.at[wid, g], snd_v)
            pltpu.sync_copy(rcv_hbm.at[wid, g], rcv_v)
            gather(0, a_v0, b_v0, sg0)
            gather(1, a_v1, b_v1, sg1)

            def half(k0, a_v, b_v, sg):
                gather_wait(a_v, b_v, sg)
                compute(a_v, b_v)
                pltpu.async_copy(a_v, agg_sh.at[snd_v.at[k0]], ss, add=True)
                scatter_wait(a_v)

                @pl.when(k0 + 2 < idx_block)
                def _():
                    gather(k0 + 2, a_v, b_v, sg)

            def pair_body(p, carry):
                k0 = 2 * p
                half(k0, a_v0, b_v0, sg0)
                half(k0 + 1, a_v1, b_v1, sg1)
                return carry

            lax.fori_loop(0, idx_block // 2, pair_body, 0, unroll=False)
            return carry0

        lax.fori_loop(0, n_blocks, block_body, 0, unroll=False)
        plsc.subcore_barrier()

        pltpu.sync_copy(agg_sh.at[own], out_hbm.at[c, own])

    return sc_edge


def kernel(variables, factors, senders, receivers, W_msg, b_msg, W_comb, b_comb):
    n_var, d = variables.shape
    e = senders.shape[0]

    w1 = W_msg[:d]
    w2 = W_msg[d:]
    wc1 = W_comb[:d]
    wc2 = W_comb[d:]
    bm = b_msg.reshape(1, d)
    bc = b_comb.reshape(1, d)

    n_pad = -(-n_var // (NS * 8)) * (NS * 8)

    var_pad = jnp.pad(variables, ((0, n_pad - n_var), (0, 0)))
    fac_pad = jnp.pad(factors, ((0, n_pad - factors.shape[0]), (0, 0)))

    blk_v = n_pad // 8
    blk_f = n_pad // 8
    grid8 = (8,)
    a_mat, b_mat = pl.pallas_call(
        _pre_mm_kernel,
        grid=grid8,
        in_specs=[
            pl.BlockSpec((blk_v, d), lambda i: (i, 0)),
            pl.BlockSpec((blk_f, d), lambda i: (i, 0)),
            pl.BlockSpec((d, d), lambda i: (0, 0)),
            pl.BlockSpec((d, d), lambda i: (0, 0)),
            pl.BlockSpec((1, d), lambda i: (0, 0)),
        ],
        out_specs=[
            pl.BlockSpec((blk_v, d), lambda i: (i, 0)),
            pl.BlockSpec((blk_f, d), lambda i: (i, 0)),
        ],
        out_shape=[
            jax.ShapeDtypeStruct((n_pad, d), jnp.float32),
            jax.ShapeDtypeStruct((n_pad, d), jnp.float32),
        ],
    )(var_pad, fac_pad, w1, w2, bm)

    chunk = 80
    per_w = e // NW
    n_chunks = per_w // chunk
    snd = senders.reshape(NW, n_chunks, chunk)
    rcv = receivers.reshape(NW, n_chunks, chunk)
    if n_chunks % 2:
        n_chunks += 1
        pad_snd = jnp.full((NW, 1, chunk), n_var, jnp.int32)
        snd = jnp.concatenate([snd, pad_snd], axis=1)
        rcv = jnp.concatenate([rcv, jnp.zeros((NW, 1, chunk), jnp.int32)], axis=1)

    idx_block = 18
    n_blocks = n_chunks // idx_block
    snd = snd.reshape(NW, n_blocks, idx_block, chunk)
    rcv = rcv.reshape(NW, n_blocks, idx_block, chunk)

    zeros = jnp.zeros((n_pad, d), jnp.float32)

    partial = _make_sc_edge_kernel(n_pad, d, chunk, n_chunks, idx_block)(
        a_mat, b_mat, snd, rcv, zeros)
    partial = partial[:, :n_var, :]

    blk = 2000
    grid = (n_var // blk,)
    row_spec = pl.BlockSpec((blk, d), lambda i: (i, 0))
    w_spec = pl.BlockSpec((d, d), lambda i: (0, 0))
    b_spec = pl.BlockSpec((1, d), lambda i: (0, 0))
    out = pl.pallas_call(
        _comb_mm_kernel,
        grid=grid,
        in_specs=[row_spec, row_spec, row_spec, w_spec, w_spec, b_spec],
        out_specs=row_spec,
        out_shape=jax.ShapeDtypeStruct((n_var, d), jnp.float32),
    )(variables, partial[0], partial[1], wc1, wc2, bc)

    return out


# speedup vs baseline: 2.8197x; 1.9879x over previous
import functools

import jax
import jax.numpy as jnp
from jax import lax
from jax.experimental import pallas as pl
from jax.experimental.pallas import tpu as pltpu
from jax.experimental.pallas import tpu_sc as plsc

NC = 2
NS = 16
NW = NC * NS
L = 16


def _pre_mm_kernel(v_ref, f_ref, w1_ref, w2_ref, b_ref, a_out, b_out):
    a_out[...] = (
        jnp.dot(v_ref[...], w1_ref[...], preferred_element_type=jnp.float32)
        + b_ref[...]
    )
    b_out[...] = jnp.dot(f_ref[...], w2_ref[...], preferred_element_type=jnp.float32)


def _comb_mm_kernel(v_ref, p0_ref, p1_ref, wc1_ref, wc2_ref, b_ref, o_ref):
    agg = p0_ref[...] + p1_ref[...]
    h = (
        jnp.dot(v_ref[...], wc1_ref[...], preferred_element_type=jnp.float32)
        + jnp.dot(agg, wc2_ref[...], preferred_element_type=jnp.float32)
        + b_ref[...]
    )
    o_ref[...] = v_ref[...] + jnp.maximum(h, 0.0)


def _make_sc_edge_kernel(n_pad, d, chunk, n_chunks, idx_block):
    rows_per_tile = n_pad // NS
    n_blocks = n_chunks // idx_block

    mesh = plsc.VectorSubcoreMesh(core_axis_name="c", subcore_axis_name="s")

    assert idx_block % 2 == 0 and n_blocks * idx_block == n_chunks

    @functools.partial(
        pl.kernel,
        out_type=jax.ShapeDtypeStruct((NC, n_pad, d), jnp.float32),
        mesh=mesh,
        scratch_types=[
            pltpu.VMEM((idx_block, chunk), jnp.int32),
            pltpu.VMEM((idx_block, chunk), jnp.int32),
            pltpu.VMEM((chunk, d), jnp.float32),
            pltpu.VMEM((chunk, d), jnp.float32),
            pltpu.VMEM_SHARED((n_pad, d), jnp.float32),
            pltpu.SemaphoreType.DMA,
            pltpu.SemaphoreType.DMA,
            pltpu.SemaphoreType.DMA,
            pltpu.SemaphoreType.DMA,
            pltpu.SemaphoreType.DMA,
        ],
    )
    def sc_edge(a_hbm, b_hbm, snd_hbm, rcv_hbm, zero_hbm, out_hbm,
                snd_v, rcv_v, a_v0, a_v1, agg_sh,
                sa0, sa1, sb0, sb1, ss):
        c = lax.axis_index("c")
        s = lax.axis_index("s")
        wid = c * NS + s
        own = pl.ds(s * rows_per_tile, rows_per_tile)

        def issue_a(k, a_v, sa):
            pltpu.async_copy(a_hbm.at[snd_v.at[k]], a_v, sa)

        def issue_b(k, a_v, sb):
            pltpu.async_copy(b_hbm.at[rcv_v.at[k]], a_v, sb, add=True)

        def wait(a_v, sem):
            pltpu.make_async_copy(a_hbm.at[pl.ds(0, chunk)], a_v, sem).wait()

        def relu(a_v):
            def row_body(i, carry2):
                for j in range(d // L):
                    sl = pl.ds(j * L, L)
                    a_v[i, sl] = jnp.maximum(a_v[i, sl], 0.0)
                return carry2

            lax.fori_loop(0, chunk, row_body, 0, unroll=4)

        def finish(k, a_v):
            relu(a_v)
            pltpu.async_copy(a_v, agg_sh.at[snd_v.at[k]], ss, add=True)
            wait(a_v, ss)

        pltpu.sync_copy(zero_hbm.at[own], agg_sh.at[own])
        plsc.subcore_barrier()

        def block_body(g, carry0):
            pltpu.sync_copy(snd_hbm.at[wid, g], snd_v)
            pltpu.sync_copy(rcv_hbm.at[wid, g], rcv_v)
            issue_a(0, a_v0, sa0)

            def body(k, x_v, sax, sbx, y_v, say, sby):
                wait(x_v, sax)
                issue_b(k, x_v, sbx)

                @pl.when(k > 0)
                def _():
                    wait(y_v, sby)
                    finish(k - 1, y_v)

                @pl.when(k + 1 < idx_block)
                def _():
                    issue_a(k + 1, y_v, say)

            def chunk_iter(k, carry):
                @pl.when(k % 2 == 0)
                def _():
                    body(k, a_v0, sa0, sb0, a_v1, sa1, sb1)

                @pl.when(k % 2 == 1)
                def _():
                    body(k, a_v1, sa1, sb1, a_v0, sa0, sb0)

                return carry

            lax.fori_loop(0, idx_block, chunk_iter, 0, unroll=False)
            wait(a_v1, sb1)
            finish(idx_block - 1, a_v1)
            return carry0

        lax.fori_loop(0, n_blocks, block_body, 0, unroll=False)
        plsc.subcore_barrier()

        pltpu.sync_copy(agg_sh.at[own], out_hbm.at[c, own])

    return sc_edge


def kernel(variables, factors, senders, receivers, W_msg, b_msg, W_comb, b_comb):
    n_var, d = variables.shape
    e = senders.shape[0]

    w1 = W_msg[:d]
    w2 = W_msg[d:]
    wc1 = W_comb[:d]
    wc2 = W_comb[d:]
    bm = b_msg.reshape(1, d)
    bc = b_comb.reshape(1, d)

    n_pad = -(-n_var // (NS * 8)) * (NS * 8)

    var_pad = jnp.pad(variables, ((0, n_pad - n_var), (0, 0)))
    fac_pad = jnp.pad(factors, ((0, n_pad - factors.shape[0]), (0, 0)))

    blk_v = n_pad // 8
    blk_f = n_pad // 8
    grid8 = (8,)
    a_mat, b_mat = pl.pallas_call(
        _pre_mm_kernel,
        grid=grid8,
        in_specs=[
            pl.BlockSpec((blk_v, d), lambda i: (i, 0)),
            pl.BlockSpec((blk_f, d), lambda i: (i, 0)),
            pl.BlockSpec((d, d), lambda i: (0, 0)),
            pl.BlockSpec((d, d), lambda i: (0, 0)),
            pl.BlockSpec((1, d), lambda i: (0, 0)),
        ],
        out_specs=[
            pl.BlockSpec((blk_v, d), lambda i: (i, 0)),
            pl.BlockSpec((blk_f, d), lambda i: (i, 0)),
        ],
        out_shape=[
            jax.ShapeDtypeStruct((n_pad, d), jnp.float32),
            jax.ShapeDtypeStruct((n_pad, d), jnp.float32),
        ],
    )(var_pad, fac_pad, w1, w2, bm)

    chunk = 80
    per_w = e // NW
    n_chunks = per_w // chunk
    snd = senders.reshape(NW, n_chunks, chunk)
    rcv = receivers.reshape(NW, n_chunks, chunk)
    if n_chunks % 2:
        n_chunks += 1
        lane = jnp.arange(NW * chunk, dtype=jnp.int32).reshape(NW, 1, chunk)
        pad_snd = n_var + (lane % (n_pad - n_var))
        pad_rcv = lane % n_pad
        snd = jnp.concatenate([snd, pad_snd], axis=1)
        rcv = jnp.concatenate([rcv, pad_rcv], axis=1)

    idx_block = 18
    n_blocks = n_chunks // idx_block
    snd = snd.reshape(NW, n_blocks, idx_block, chunk)
    rcv = rcv.reshape(NW, n_blocks, idx_block, chunk)

    zeros = jnp.zeros((n_pad, d), jnp.float32)

    partial = _make_sc_edge_kernel(n_pad, d, chunk, n_chunks, idx_block)(
        a_mat, b_mat, snd, rcv, zeros)
    partial = partial[:, :n_var, :]

    blk = 2000
    grid = (n_var // blk,)
    row_spec = pl.BlockSpec((blk, d), lambda i: (i, 0))
    w_spec = pl.BlockSpec((d, d), lambda i: (0, 0))
    b_spec = pl.BlockSpec((1, d), lambda i: (0, 0))
    out = pl.pallas_call(
        _comb_mm_kernel,
        grid=grid,
        in_specs=[row_spec, row_spec, row_spec, w_spec, w_spec, b_spec],
        out_specs=row_spec,
        out_shape=jax.ShapeDtypeStruct((n_var, d), jnp.float32),
    )(variables, partial[0], partial[1], wc1, wc2, bc)

    return out
